# baseline (speedup 1.0000x reference)
import jax
import jax.numpy as jnp
from jax import lax
from jax.experimental import pallas as pl
from jax.experimental.pallas import tpu as pltpu
from jax.experimental.pallas import tpu_sc as plsc

N_NODES = 10000
N_PAD = 10240
N_EDGES = 320000
DIM = 128

NC = 2
NS = 16
NW = NC * NS

CK = 128
EDGES_PER_TILE = N_PAD
E_PAD = NW * EDGES_PER_TILE
CHUNKS_PER_TILE = EDGES_PER_TILE // CK
U = 2
NGROUPS = CHUNKS_PER_TILE // U
ROWS_PER_TILE = N_PAD // NS
RB = CK

_MESH = plsc.VectorSubcoreMesh(
    core_axis_name="c", subcore_axis_name="s", num_cores=NC, num_subcores=NS)


def _fill(ref, n_rows, value):
  def body(i, carry):
    for k in range(DIM // 16):
      ref[i, pl.ds(k * 16, 16)] = jnp.full((16,), value, jnp.float32)
    return carry
  lax.fori_loop(0, n_rows, body, 0)


def _make_sc_pass(compute_deg: bool):
  out_type = [jax.ShapeDtypeStruct((NC, N_PAD, DIM), jnp.float32)]
  if compute_deg:
    out_type.append(jax.ShapeDtypeStruct((NC, N_PAD, DIM), jnp.float32))

  scratch = [
      pltpu.VMEM((2, U, 2, CK), jnp.int32),
      pltpu.VMEM((U, CK, DIM), jnp.float32),
      pltpu.VMEM_SHARED((N_PAD, DIM), jnp.float32),
  ] + [pltpu.SemaphoreType.DMA] * (2 * U + 3)

  def body(table_hbm, ei_hbm, *rest):
    if compute_deg:
      acc_out, deg_out = rest[0], rest[1]
      rest = rest[2:]
    else:
      acc_out, deg_out = rest[0], None
      rest = rest[1:]
    idxg2, rowsg, acc_sh = rest[:3]
    sems = rest[3:]
    gsem, ssem = sems[:U], sems[U:2 * U]
    isem = sems[2 * U]
    wsem = sems[2 * U + 1:2 * U + 3]

    c = lax.axis_index("c")
    s = lax.axis_index("s")
    w = c * NS + s
    row0 = s * ROWS_PER_TILE
    ei_w = ei_hbm.at[w]

    def zero_stripe():
      _fill(rowsg.at[0], CK, 0.0)
      for k in range(ROWS_PER_TILE // RB):
        pltpu.sync_copy(rowsg.at[0], acc_sh.at[pl.ds(row0 + k * RB, RB)])

    def epilogue(out_hbm):
      wds = [None, None]
      for k in range(ROWS_PER_TILE // RB):
        b = k % 2
        if wds[b] is not None:
          wds[b].wait()
        r = row0 + k * RB
        pltpu.sync_copy(acc_sh.at[pl.ds(r, RB)], rowsg.at[b])
        wds[b] = pltpu.async_copy(rowsg.at[b],
                                  out_hbm.at[c].at[pl.ds(r, RB)], wsem[b])
      for d in wds:
        d.wait()

    def scatter_loop(do_gather):
      src_buf = lambda b: rowsg.at[b] if do_gather else rowsg.at[0]
      pltpu.async_copy(ei_w.at[pl.ds(0, U)], idxg2.at[0], isem)

      def step(g, carry):
        slot = lax.rem(g, 2)
        pltpu.make_async_copy(ei_w.at[pl.ds(g * U, U)], idxg2.at[slot],
                              isem).wait()
        idxg = idxg2.at[slot]
        gds = []
        for b in range(U):
          @pl.when(g > 0)
          def _(b=b):
            pltpu.make_async_copy(src_buf(b), acc_sh.at[idxg.at[b].at[1]],
                                  ssem[b]).wait()
          if do_gather:
            gds.append(pltpu.async_copy(table_hbm.at[idxg.at[b].at[0]],
                                        rowsg.at[b], gsem[b]))
        @pl.when(g + 1 < NGROUPS)
        def _():
          pltpu.async_copy(ei_w.at[pl.ds((g + 1) * U, U)],
                           idxg2.at[lax.rem(g + 1, 2)], isem)
        for b in range(U):
          if do_gather:
            gds[b].wait()
          pltpu.async_copy(src_buf(b), acc_sh.at[idxg.at[b].at[1]],
                           ssem[b], add=True)
        return carry
      lax.fori_loop(0, NGROUPS, step, 0)

      for b in range(U):
        pltpu.make_async_copy(src_buf(b), acc_sh.at[idxg2.at[0].at[b].at[1]],
                              ssem[b]).wait()
      for j in range(NGROUPS * U, CHUNKS_PER_TILE):
        pltpu.sync_copy(ei_w.at[pl.ds(j, 1)], idxg2.at[0].at[pl.ds(0, 1)])
        lidx = idxg2.at[0].at[0]
        if do_gather:
          pltpu.async_copy(table_hbm.at[lidx.at[0]], rowsg.at[0],
                           gsem[0]).wait()
        pltpu.sync_copy(src_buf(0), acc_sh.at[lidx.at[1]], add=True)

    if compute_deg:
      zero_stripe()
      plsc.subcore_barrier()
      _fill(rowsg.at[0], CK, 1.0)
      scatter_loop(do_gather=False)
      plsc.subcore_barrier()
      epilogue(deg_out)

    zero_stripe()
    plsc.subcore_barrier()
    scatter_loop(do_gather=True)
    plsc.subcore_barrier()
    epilogue(acc_out)

  return pl.kernel(body, out_type=out_type, mesh=_MESH,
                   scratch_types=scratch, name="sc_sage_agg")


_sc_pass1 = _make_sc_pass(compute_deg=True)
_sc_pass2 = _make_sc_pass(compute_deg=False)


def _tc_body(x_ref, p_ref, d0_ref, d1_ref, ws_ref, wn_ref,
             b_ref, g_ref, be_ref, o_ref):
  deg = d0_ref[...] + d1_ref[...]
  degc = jnp.maximum(deg, 1.0)
  mean = (p_ref[0, :N_NODES, :] + p_ref[1, :N_NODES, :]) / degc
  h = (jnp.dot(x_ref[...], ws_ref[...], preferred_element_type=jnp.float32)
       + jnp.dot(mean, wn_ref[...], preferred_element_type=jnp.float32)
       + b_ref[...])
  m = jnp.mean(h, axis=0, keepdims=True)
  v = jnp.mean((h - m) * (h - m), axis=0, keepdims=True)
  hn = (h - m) * jax.lax.rsqrt(v + 1e-5) * g_ref[...] + be_ref[...]
  o_ref[...] = jnp.where(hn >= 0.0, hn, 0.01 * hn)


def _tc_layer(x, p, d0, d1, w_self, w_neigh, b, g, be):
  return pl.pallas_call(
      _tc_body,
      out_shape=jax.ShapeDtypeStruct((N_NODES, DIM), jnp.float32),
  )(x, p, d0, d1, w_self, w_neigh,
    b.reshape(1, DIM), g.reshape(1, DIM), be.reshape(1, DIM))


def kernel(x, edge_index, W1_self, W1_neigh, b1, g1, be1,
           W2_self, W2_neigh, b2, g2, be2):
  pad = E_PAD - N_EDGES
  pad_i = jnp.arange(pad, dtype=jnp.int32)
  src_p = jnp.concatenate([edge_index[0].astype(jnp.int32),
                           pad_i % N_NODES])
  dst_p = jnp.concatenate([edge_index[1].astype(jnp.int32),
                           N_NODES + pad_i % (N_PAD - N_NODES)])
  ei = jnp.stack([
      src_p.reshape(NW, CHUNKS_PER_TILE, CK),
      dst_p.reshape(NW, CHUNKS_PER_TILE, CK),
  ], axis=2)

  acc1, degp = _sc_pass1(x, ei)
  d0 = degp[0, :N_NODES, 0:1]
  d1 = degp[1, :N_NODES, 0:1]
  h1 = _tc_layer(x, acc1, d0, d1, W1_self, W1_neigh, b1, g1, be1)
  acc2, = _sc_pass2(h1, ei)
  h2 = _tc_layer(h1, acc2, d0, d1, W2_self, W2_neigh, b2, g2, be2)
  return h2


# speedup vs baseline: 1.1529x; 1.1529x over previous
import jax
import jax.numpy as jnp
from jax import lax
from jax.experimental import pallas as pl
from jax.experimental.pallas import tpu as pltpu
from jax.experimental.pallas import tpu_sc as plsc

N_NODES = 10000
N_PAD = 10240
N_EDGES = 320000
DIM = 128

NC = 2
NS = 16
NW = NC * NS

CK = 80
EDGES_PER_TILE = N_EDGES // NW
CHUNKS_PER_TILE = EDGES_PER_TILE // CK
U = 4
NGROUPS = CHUNKS_PER_TILE // U
ROWS_PER_TILE = N_PAD // NS
RB = CK

_MESH = plsc.VectorSubcoreMesh(
    core_axis_name="c", subcore_axis_name="s", num_cores=NC, num_subcores=NS)


def _fill(ref, n_rows, value):
  def body(i, carry):
    for k in range(DIM // 16):
      ref[i, pl.ds(k * 16, 16)] = jnp.full((16,), value, jnp.float32)
    return carry
  lax.fori_loop(0, n_rows, body, 0)


def _make_sc_pass(compute_deg: bool):
  out_type = [jax.ShapeDtypeStruct((NC, N_PAD, DIM), jnp.float32)]
  if compute_deg:
    out_type.append(jax.ShapeDtypeStruct((NC, N_PAD, DIM), jnp.float32))

  scratch = [
      pltpu.VMEM((2, U, 2, CK), jnp.int32),
      pltpu.VMEM((U, CK, DIM), jnp.float32),
      pltpu.VMEM_SHARED((N_PAD, DIM), jnp.float32),
  ] + [pltpu.SemaphoreType.DMA] * (2 * U + 3)

  def body(table_hbm, ei_hbm, *rest):
    if compute_deg:
      acc_out, deg_out = rest[0], rest[1]
      rest = rest[2:]
    else:
      acc_out, deg_out = rest[0], None
      rest = rest[1:]
    idxg2, rowsg, acc_sh = rest[:3]
    sems = rest[3:]
    gsem, ssem = sems[:U], sems[U:2 * U]
    isem = sems[2 * U]
    wsem = sems[2 * U + 1:2 * U + 3]

    c = lax.axis_index("c")
    s = lax.axis_index("s")
    w = c * NS + s
    row0 = s * ROWS_PER_TILE
    ei_w = ei_hbm.at[w]

    def zero_stripe():
      _fill(rowsg.at[0], CK, 0.0)
      for k in range(ROWS_PER_TILE // RB):
        pltpu.sync_copy(rowsg.at[0], acc_sh.at[pl.ds(row0 + k * RB, RB)])

    def epilogue(out_hbm):
      wds = [None, None]
      for k in range(ROWS_PER_TILE // RB):
        b = k % 2
        if wds[b] is not None:
          wds[b].wait()
        r = row0 + k * RB
        pltpu.sync_copy(acc_sh.at[pl.ds(r, RB)], rowsg.at[b])
        wds[b] = pltpu.async_copy(rowsg.at[b],
                                  out_hbm.at[c].at[pl.ds(r, RB)], wsem[b])
      for d in wds:
        d.wait()

    def scatter_loop(do_gather):
      src_buf = lambda b: rowsg.at[b] if do_gather else rowsg.at[0]
      pltpu.async_copy(ei_w.at[pl.ds(0, U)], idxg2.at[0], isem)

      def step(g, carry):
        slot = lax.rem(g, 2)
        pltpu.make_async_copy(ei_w.at[pl.ds(g * U, U)], idxg2.at[slot],
                              isem).wait()
        idxg = idxg2.at[slot]
        gds = []
        for b in range(U):
          @pl.when(g > 0)
          def _(b=b):
            pltpu.make_async_copy(src_buf(b), acc_sh.at[idxg.at[b].at[1]],
                                  ssem[b]).wait()
          if do_gather:
            gds.append(pltpu.async_copy(table_hbm.at[idxg.at[b].at[0]],
                                        rowsg.at[b], gsem[b]))
        @pl.when(g + 1 < NGROUPS)
        def _():
          pltpu.async_copy(ei_w.at[pl.ds((g + 1) * U, U)],
                           idxg2.at[lax.rem(g + 1, 2)], isem)
        for b in range(U):
          if do_gather:
            gds[b].wait()
          pltpu.async_copy(src_buf(b), acc_sh.at[idxg.at[b].at[1]],
                           ssem[b], add=True)
        return carry
      lax.fori_loop(0, NGROUPS, step, 0)

      for b in range(U):
        pltpu.make_async_copy(src_buf(b), acc_sh.at[idxg2.at[0].at[b].at[1]],
                              ssem[b]).wait()
      for j in range(NGROUPS * U, CHUNKS_PER_TILE):
        pltpu.sync_copy(ei_w.at[pl.ds(j, 1)], idxg2.at[0].at[pl.ds(0, 1)])
        lidx = idxg2.at[0].at[0]
        if do_gather:
          pltpu.async_copy(table_hbm.at[lidx.at[0]], rowsg.at[0],
                           gsem[0]).wait()
        pltpu.sync_copy(src_buf(0), acc_sh.at[lidx.at[1]], add=True)

    if compute_deg:
      zero_stripe()
      plsc.subcore_barrier()
      _fill(rowsg.at[0], CK, 1.0)
      scatter_loop(do_gather=False)
      plsc.subcore_barrier()
      epilogue(deg_out)

    zero_stripe()
    plsc.subcore_barrier()
    scatter_loop(do_gather=True)
    plsc.subcore_barrier()
    epilogue(acc_out)

  return pl.kernel(body, out_type=out_type, mesh=_MESH,
                   scratch_types=scratch, name="sc_sage_agg")


_sc_pass1 = _make_sc_pass(compute_deg=True)
_sc_pass2 = _make_sc_pass(compute_deg=False)


def _tc_body(x_ref, p_ref, d0_ref, d1_ref, ws_ref, wn_ref,
             b_ref, g_ref, be_ref, o_ref):
  deg = d0_ref[...] + d1_ref[...]
  degc = jnp.maximum(deg, 1.0)
  mean = (p_ref[0, :N_NODES, :] + p_ref[1, :N_NODES, :]) / degc
  h = (jnp.dot(x_ref[...], ws_ref[...], preferred_element_type=jnp.float32)
       + jnp.dot(mean, wn_ref[...], preferred_element_type=jnp.float32)
       + b_ref[...])
  m = jnp.mean(h, axis=0, keepdims=True)
  v = jnp.mean((h - m) * (h - m), axis=0, keepdims=True)
  hn = (h - m) * jax.lax.rsqrt(v + 1e-5) * g_ref[...] + be_ref[...]
  o_ref[...] = jnp.where(hn >= 0.0, hn, 0.01 * hn)


def _tc_layer(x, p, d0, d1, w_self, w_neigh, b, g, be):
  return pl.pallas_call(
      _tc_body,
      out_shape=jax.ShapeDtypeStruct((N_NODES, DIM), jnp.float32),
  )(x, p, d0, d1, w_self, w_neigh,
    b.reshape(1, DIM), g.reshape(1, DIM), be.reshape(1, DIM))


def kernel(x, edge_index, W1_self, W1_neigh, b1, g1, be1,
           W2_self, W2_neigh, b2, g2, be2):
  ei = jnp.stack([
      edge_index[0].astype(jnp.int32).reshape(NW, CHUNKS_PER_TILE, CK),
      edge_index[1].astype(jnp.int32).reshape(NW, CHUNKS_PER_TILE, CK),
  ], axis=2)

  acc1, degp = _sc_pass1(x, ei)
  d0 = degp[0, :N_NODES, 0:1]
  d1 = degp[1, :N_NODES, 0:1]
  h1 = _tc_layer(x, acc1, d0, d1, W1_self, W1_neigh, b1, g1, be1)
  acc2, = _sc_pass2(h1, ei)
  h2 = _tc_layer(h1, acc2, d0, d1, W2_self, W2_neigh, b2, g2, be2)
  return h2
